# baseline (speedup 1.0000x reference)
import jax
import jax.numpy as jnp
from jax import lax
from jax.experimental import pallas as pl
from jax.experimental.pallas import tpu as pltpu

_E = 16
_TEMP = 1.5
_CAP = 96


def _reduce_kernel(x_ref, w_ref, out_ref):
    x = x_ref[...]
    pooled = jnp.sum(x, axis=2) * (1.0 / x.shape[2])
    out_ref[...] = jnp.dot(pooled, w_ref[...],
                           preferred_element_type=jnp.float32)


def _stats_kernel(lr_ref, logits_ref, ep_ref, eidx_ref,
                  std_ref, z_ref, aux_ref, div_ref):
    lr = lr_ref[...]
    n, e = lr.shape
    mean = jnp.mean(lr)
    std_ref[...] = jnp.sqrt(jnp.mean((lr - mean) ** 2)).reshape(1, 1)

    l = jnp.clip(lr, -10.0, 10.0) / _TEMP
    logits_ref[...] = l

    m = jnp.max(l, axis=1, keepdims=True)
    se = jnp.sum(jnp.exp(l - m), axis=1, keepdims=True)
    lse = m + jnp.log(se)
    z_ref[...] = jnp.mean(lse * lse).reshape(1, 1)

    norm = jnp.maximum(jnp.sqrt(jnp.sum(l * l, axis=0, keepdims=True)),
                       1e-12)
    ln = l / norm
    rows = [jnp.sum(ln * ln[:, a:a + 1], axis=0, keepdims=True)
            for a in range(_E)]
    corr = jnp.concatenate(rows, axis=0)
    ia = lax.broadcasted_iota(jnp.int32, (_E, _E), 0)
    ib = lax.broadcasted_iota(jnp.int32, (_E, _E), 1)
    off = jnp.where(ia == ib, 0.0, corr)
    div_ref[...] = (jnp.sum(off * off) / (_E * (_E - 1))).reshape(1, 1)

    pe = jnp.exp(l - m)
    p = pe / se
    ep = jnp.max(p, axis=1, keepdims=True)
    ep_ref[...] = ep
    it = lax.broadcasted_iota(jnp.int32, (n, e), 1)
    eidx = jnp.min(jnp.where(p == ep, it, e), axis=1, keepdims=True)
    eidx_ref[...] = eidx

    onehot = (it == eidx).astype(jnp.float32)
    f = jnp.mean(onehot, axis=0, keepdims=True)
    pm = jnp.mean(p, axis=0, keepdims=True)
    aux_ref[...] = (jnp.sum(f * pm) * _E).reshape(1, 1)


def _route_kernel(ep_col_ref, eidx_col_ref, ep_row_ref, eidx_row_ref,
                  dispatch_ref, combine_ref):
    epc = ep_col_ref[...]
    eic = eidx_col_ref[...]
    epr = ep_row_ref[...]
    eir = eidx_row_ref[...]
    bi = epc.shape[0]
    n = epr.shape[1]

    jrow = lax.broadcasted_iota(jnp.int32, (1, n), 1)
    icol = (pl.program_id(0) * bi
            + lax.broadcasted_iota(jnp.int32, (bi, 1), 0))
    same = eir == eic
    beats = (epr > epc) | ((epr == epc) & (jrow < icol))
    rank = jnp.sum((same & beats).astype(jnp.int32), axis=1,
                   keepdims=True)
    within = rank < _CAP
    gate = jnp.where(within, epc, 0.0)

    e3 = lax.broadcasted_iota(jnp.int32, (bi, _E, _CAP), 1)
    c3 = lax.broadcasted_iota(jnp.int32, (bi, _E, _CAP), 2)
    hit = ((e3 == eic[:, :, None]) & (c3 == rank[:, :, None])
           & within[:, :, None])
    combine_ref[...] = jnp.where(hit, gate[:, :, None], 0.0)
    dispatch_ref[...] = hit


def kernel(X, W_gate, current_epoch):
    n, c, h, w = X.shape
    hw = h * w
    x3 = X.reshape(n, c, hw)

    bt = 16
    logits_raw = pl.pallas_call(
        _reduce_kernel,
        grid=(n // bt,),
        in_specs=[
            pl.BlockSpec((bt, c, hw), lambda i: (i, 0, 0)),
            pl.BlockSpec((c, _E), lambda i: (0, 0)),
        ],
        out_specs=pl.BlockSpec((bt, _E), lambda i: (i, 0)),
        out_shape=jax.ShapeDtypeStruct((n, _E), jnp.float32),
        compiler_params=pltpu.CompilerParams(
            dimension_semantics=("arbitrary",)),
    )(x3, W_gate)

    scalar = jax.ShapeDtypeStruct((1, 1), jnp.float32)
    logits, ep, eidx, stdv, z, aux, div = pl.pallas_call(
        _stats_kernel,
        out_shape=(
            jax.ShapeDtypeStruct((n, _E), jnp.float32),
            jax.ShapeDtypeStruct((n, 1), jnp.float32),
            jax.ShapeDtypeStruct((n, 1), jnp.int32),
            scalar, scalar, scalar, scalar,
        ),
    )(logits_raw)

    ep_row = ep.reshape(1, n)
    eidx_row = eidx.reshape(1, n)

    bi = 256
    dispatch, combine = pl.pallas_call(
        _route_kernel,
        grid=(n // bi,),
        in_specs=[
            pl.BlockSpec((bi, 1), lambda i: (i, 0)),
            pl.BlockSpec((bi, 1), lambda i: (i, 0)),
            pl.BlockSpec((1, n), lambda i: (0, 0)),
            pl.BlockSpec((1, n), lambda i: (0, 0)),
        ],
        out_specs=(
            pl.BlockSpec((bi, _E, _CAP), lambda i: (i, 0, 0)),
            pl.BlockSpec((bi, _E, _CAP), lambda i: (i, 0, 0)),
        ),
        out_shape=(
            jax.ShapeDtypeStruct((n, _E, _CAP), jnp.bool_),
            jax.ShapeDtypeStruct((n, _E, _CAP), jnp.float32),
        ),
        compiler_params=pltpu.CompilerParams(
            dimension_semantics=("arbitrary",)),
    )(ep, eidx, ep_row, eidx_row)

    return (dispatch, combine, z[0, 0], aux[0, 0], div[0, 0],
            stdv[0, 0], logits)


# speedup vs baseline: 1.0125x; 1.0125x over previous
import jax
import jax.numpy as jnp
from jax import lax
from jax.experimental import pallas as pl
from jax.experimental.pallas import tpu as pltpu

_E = 16
_TEMP = 1.5
_CAP = 96


def _reduce_kernel(x_ref, w_ref, out_ref):
    x = x_ref[...]
    pooled = jnp.sum(x, axis=2) * (1.0 / x.shape[2])
    out_ref[...] = jnp.dot(pooled, w_ref[...],
                           preferred_element_type=jnp.float32)


def _stats_kernel(lr_ref, logits_ref, ep_ref, eidx_ref,
                  std_ref, z_ref, aux_ref, div_ref):
    lr = lr_ref[...]
    n, e = lr.shape
    mean = jnp.mean(lr)
    std_ref[...] = jnp.sqrt(jnp.mean((lr - mean) ** 2)).reshape(1, 1)

    l = jnp.clip(lr, -10.0, 10.0) / _TEMP
    logits_ref[...] = l

    m = jnp.max(l, axis=1, keepdims=True)
    se = jnp.sum(jnp.exp(l - m), axis=1, keepdims=True)
    lse = m + jnp.log(se)
    z_ref[...] = jnp.mean(lse * lse).reshape(1, 1)

    norm = jnp.maximum(jnp.sqrt(jnp.sum(l * l, axis=0, keepdims=True)),
                       1e-12)
    ln = l / norm
    rows = [jnp.sum(ln * ln[:, a:a + 1], axis=0, keepdims=True)
            for a in range(_E)]
    corr = jnp.concatenate(rows, axis=0)
    ia = lax.broadcasted_iota(jnp.int32, (_E, _E), 0)
    ib = lax.broadcasted_iota(jnp.int32, (_E, _E), 1)
    off = jnp.where(ia == ib, 0.0, corr)
    div_ref[...] = (jnp.sum(off * off) / (_E * (_E - 1))).reshape(1, 1)

    pe = jnp.exp(l - m)
    p = pe / se
    ep = jnp.max(p, axis=1, keepdims=True)
    ep_ref[...] = ep
    it = lax.broadcasted_iota(jnp.int32, (n, e), 1)
    eidx = jnp.min(jnp.where(p == ep, it, e), axis=1, keepdims=True)
    eidx_ref[...] = eidx

    onehot = (it == eidx).astype(jnp.float32)
    f = jnp.mean(onehot, axis=0, keepdims=True)
    pm = jnp.mean(p, axis=0, keepdims=True)
    aux_ref[...] = (jnp.sum(f * pm) * _E).reshape(1, 1)


def _route_kernel(ep_col_ref, eidx_col_ref, ep_row_ref, eidx_row_ref,
                  dispatch_ref, combine_ref):
    epc = ep_col_ref[...]
    eic = eidx_col_ref[...]
    epr = ep_row_ref[...]
    eir = eidx_row_ref[...]
    bi = epc.shape[0]
    n = epr.shape[1]

    jrow = lax.broadcasted_iota(jnp.int32, (1, n), 1)
    icol = (pl.program_id(0) * bi
            + lax.broadcasted_iota(jnp.int32, (bi, 1), 0))
    same = eir == eic
    beats = (epr > epc) | ((epr == epc) & (jrow < icol))
    rank = jnp.sum((same & beats).astype(jnp.int32), axis=1,
                   keepdims=True)
    within = rank < _CAP
    gate = jnp.where(within, epc, 0.0)

    e3 = lax.broadcasted_iota(jnp.int32, (bi, _E, _CAP), 1)
    c3 = lax.broadcasted_iota(jnp.int32, (bi, _E, _CAP), 2)
    hit = ((e3 == eic[:, :, None]) & (c3 == rank[:, :, None])
           & within[:, :, None])
    combine_ref[...] = jnp.where(hit, gate[:, :, None], 0.0)
    dispatch_ref[...] = hit


def kernel(X, W_gate, current_epoch):
    n, c, h, w = X.shape
    hw = h * w
    x3 = X.reshape(n, c, hw)

    bt = 32
    logits_raw = pl.pallas_call(
        _reduce_kernel,
        grid=(n // bt,),
        in_specs=[
            pl.BlockSpec((bt, c, hw), lambda i: (i, 0, 0)),
            pl.BlockSpec((c, _E), lambda i: (0, 0)),
        ],
        out_specs=pl.BlockSpec((bt, _E), lambda i: (i, 0)),
        out_shape=jax.ShapeDtypeStruct((n, _E), jnp.float32),
        compiler_params=pltpu.CompilerParams(
            dimension_semantics=("parallel",)),
    )(x3, W_gate)

    scalar = jax.ShapeDtypeStruct((1, 1), jnp.float32)
    logits, ep, eidx, stdv, z, aux, div = pl.pallas_call(
        _stats_kernel,
        out_shape=(
            jax.ShapeDtypeStruct((n, _E), jnp.float32),
            jax.ShapeDtypeStruct((n, 1), jnp.float32),
            jax.ShapeDtypeStruct((n, 1), jnp.int32),
            scalar, scalar, scalar, scalar,
        ),
    )(logits_raw)

    ep_row = ep.reshape(1, n)
    eidx_row = eidx.reshape(1, n)

    bi = 256
    dispatch, combine = pl.pallas_call(
        _route_kernel,
        grid=(n // bi,),
        in_specs=[
            pl.BlockSpec((bi, 1), lambda i: (i, 0)),
            pl.BlockSpec((bi, 1), lambda i: (i, 0)),
            pl.BlockSpec((1, n), lambda i: (0, 0)),
            pl.BlockSpec((1, n), lambda i: (0, 0)),
        ],
        out_specs=(
            pl.BlockSpec((bi, _E, _CAP), lambda i: (i, 0, 0)),
            pl.BlockSpec((bi, _E, _CAP), lambda i: (i, 0, 0)),
        ),
        out_shape=(
            jax.ShapeDtypeStruct((n, _E, _CAP), jnp.bool_),
            jax.ShapeDtypeStruct((n, _E, _CAP), jnp.float32),
        ),
        compiler_params=pltpu.CompilerParams(
            dimension_semantics=("arbitrary",)),
    )(ep, eidx, ep_row, eidx_row)

    return (dispatch, combine, z[0, 0], aux[0, 0], div[0, 0],
            stdv[0, 0], logits)
